# Initial kernel scaffold; baseline (speedup 1.0000x reference)
import jax
import jax.numpy as jnp
from jax.experimental import pallas as pl


def kernel(x, edge_index, W1_l, b1_l, W1_r, W2_l, b2_l, W2_r):
    raise NotImplementedError("write your pallas kernel here")


# speedup vs baseline: 15.2399x; 15.2399x over previous
import functools

import jax
import jax.numpy as jnp
from jax import lax
from jax.experimental import pallas as pl
from jax.experimental.pallas import tpu as pltpu
from jax.experimental.pallas import tpu_sc as plsc

N = 100000
E = 1600000
D = 16
H = 32

NSUB = 16
NCORE = 2
NW = NSUB * NCORE
N8 = 100096
RPS = N8 // NSUB
EP = 1605632
EROWS = EP // 128
ROWS_W = EROWS // NW
K = 8
NCH = ROWS_W // K
ZR = RPS // NSUB


def _sc_agg_body(table, sr70, dst70, out_acc,
                 acc_sh, idx_s, idx_d, rows_v, z2d, sem):
    c = lax.axis_index("c")
    s = lax.axis_index("s")
    wid = s * NCORE + c

    def _zfill(i, carry):
        z2d[i, :] = jnp.zeros((16,), jnp.float32)
        return carry
    lax.fori_loop(0, ZR, _zfill, 0)
    base = s * RPS
    def _zcopy(k, carry):
        pltpu.sync_copy(z2d, acc_sh.at[pl.ds(base + k * ZR, ZR)])
        return carry
    lax.fori_loop(0, NSUB, _zcopy, 0)
    plsc.subcore_barrier()

    row0 = wid * ROWS_W
    def _chunk(i, carry):
        rbase = row0 + i * K
        pltpu.sync_copy(sr70.at[pl.ds(rbase, K)], idx_s)
        pltpu.sync_copy(dst70.at[pl.ds(rbase, K)], idx_d)
        cps = [
            pltpu.async_copy(table.at[idx_s.at[j]],
                             rows_v.at[pl.ds(j * 128, 128)], sem)
            for j in range(K)
        ]
        for cp in cps:
            cp.wait()
        for j in range(K):
            pltpu.sync_copy(rows_v.at[pl.ds(j * 128, 128)],
                            acc_sh.at[idx_d.at[j]], add=True)
        return carry
    lax.fori_loop(0, NCH, _chunk, 0)
    plsc.subcore_barrier()

    pltpu.sync_copy(acc_sh.at[pl.ds(base, RPS)],
                    out_acc.at[pl.ds(c * N8 + base, RPS)])


def _sc_deg_body(dst70, out_deg, deg_sh, idx_d, ones_v, z1, sem):
    c = lax.axis_index("c")
    s = lax.axis_index("s")
    wid = s * NCORE + c

    def _zfill(i, carry):
        z1[pl.ds(i * 16, 16)] = jnp.zeros((16,), jnp.float32)
        return carry
    lax.fori_loop(0, ZR, _zfill, 0)
    base = s * RPS
    pltpu.sync_copy(z1, deg_sh.at[pl.ds(base, RPS)])
    for k in range(8):
        ones_v[pl.ds(k * 16, 16)] = jnp.ones((16,), jnp.float32)
    plsc.subcore_barrier()

    row0 = wid * ROWS_W
    def _chunk(i, carry):
        rbase = row0 + i * K
        pltpu.sync_copy(dst70.at[pl.ds(rbase, K)], idx_d)
        for j in range(K):
            pltpu.sync_copy(ones_v, deg_sh.at[idx_d.at[j]], add=True)
        return carry
    lax.fori_loop(0, NCH, _chunk, 0)
    plsc.subcore_barrier()

    pltpu.sync_copy(deg_sh.at[pl.ds(base, RPS)],
                    out_deg.at[pl.ds(c * N8 + base, RPS)])


_MESH = plsc.VectorSubcoreMesh(core_axis_name="c", subcore_axis_name="s")
_SC_PARAMS = pltpu.CompilerParams(use_tc_tiling_on_sc=False)

_agg = pl.kernel(
    _sc_agg_body,
    compiler_params=_SC_PARAMS,
    out_type=jax.ShapeDtypeStruct((NCORE * N8, D), jnp.float32),
    mesh=_MESH,
    scratch_types=[
        pltpu.VMEM_SHARED((N8, D), jnp.float32),
        pltpu.VMEM((K, 128), jnp.int32),
        pltpu.VMEM((K, 128), jnp.int32),
        pltpu.VMEM((K * 128, D), jnp.float32),
        pltpu.VMEM((ZR, D), jnp.float32),
        pltpu.SemaphoreType.DMA,
    ],
)

_deg_count = pl.kernel(
    _sc_deg_body,
    compiler_params=_SC_PARAMS,
    out_type=jax.ShapeDtypeStruct((NCORE * N8,), jnp.float32),
    mesh=_MESH,
    scratch_types=[
        pltpu.VMEM_SHARED((N8,), jnp.float32),
        pltpu.VMEM((K, 128), jnp.int32),
        pltpu.VMEM((128,), jnp.float32),
        pltpu.VMEM((RPS,), jnp.float32),
        pltpu.SemaphoreType.DMA,
    ],
)

R = 3128
G = N8 // R


def _dense1_body(p_ref, deg_ref, x_ref, w1l_ref, b1_ref, w1r_ref,
                 w2l_ref, w2r_ref, b2_ref, g_ref, r_ref):
    p = p_ref[0] + p_ref[1]
    d = deg_ref[0] + deg_ref[1]
    mean = p / jnp.clip(d, 1.0)
    h = jnp.maximum(
        mean @ w1l_ref[...] + b1_ref[...] + x_ref[...] @ w1r_ref[...], 0.0)
    g_ref[...] = h @ w2l_ref[...]
    r_ref[...] = h @ w2r_ref[...] + b2_ref[...]


_dense1 = pl.pallas_call(
    _dense1_body,
    grid=(G,),
    in_specs=[
        pl.BlockSpec((NCORE, R, D), lambda i: (0, i, 0)),
        pl.BlockSpec((NCORE, R, 1), lambda i: (0, i, 0)),
        pl.BlockSpec((R, D), lambda i: (i, 0)),
        pl.BlockSpec((D, H), lambda i: (0, 0)),
        pl.BlockSpec((1, H), lambda i: (0, 0)),
        pl.BlockSpec((D, H), lambda i: (0, 0)),
        pl.BlockSpec((H, D), lambda i: (0, 0)),
        pl.BlockSpec((H, D), lambda i: (0, 0)),
        pl.BlockSpec((1, D), lambda i: (0, 0)),
    ],
    out_specs=[
        pl.BlockSpec((R, D), lambda i: (i, 0)),
        pl.BlockSpec((R, D), lambda i: (i, 0)),
    ],
    out_shape=[
        jax.ShapeDtypeStruct((N8, D), jnp.float32),
        jax.ShapeDtypeStruct((N8, D), jnp.float32),
    ],
)


def _dense2_body(p_ref, deg_ref, r_ref, o_ref):
    p = p_ref[0] + p_ref[1]
    d = deg_ref[0] + deg_ref[1]
    o_ref[...] = p / jnp.clip(d, 1.0) + r_ref[...]


_dense2 = pl.pallas_call(
    _dense2_body,
    grid=(G,),
    in_specs=[
        pl.BlockSpec((NCORE, R, D), lambda i: (0, i, 0)),
        pl.BlockSpec((NCORE, R, 1), lambda i: (0, i, 0)),
        pl.BlockSpec((R, D), lambda i: (i, 0)),
    ],
    out_specs=pl.BlockSpec((R, D), lambda i: (i, 0)),
    out_shape=jax.ShapeDtypeStruct((N8, D), jnp.float32),
)


def kernel(x, edge_index, W1_l, b1_l, W1_r, W2_l, b2_l, W2_r):
    pad = EP - E
    srcp = jnp.concatenate(
        [edge_index[0], jnp.zeros((pad,), jnp.int32)]).reshape(EROWS, 128)
    dstp = jnp.concatenate(
        [edge_index[1], jnp.full((pad,), N, jnp.int32)]).reshape(EROWS, 128)
    x_pad = jnp.pad(x, ((0, N8 - N), (0, 0)))

    accf = _agg(x_pad, srcp, dstp)
    degf = _deg_count(dstp)
    p1 = accf.reshape(NCORE, N8, D)
    degr = degf.reshape(NCORE, N8, 1)
    g, r = _dense1(p1, degr, x_pad, W1_l, b1_l.reshape(1, H), W1_r,
                   W2_l, W2_r, b2_l.reshape(1, D))
    acc2f = _agg(g, srcp, dstp)
    p2 = acc2f.reshape(NCORE, N8, D)
    out = _dense2(p2, degr, r)
    return out[:N]


# speedup vs baseline: 16.1598x; 1.0604x over previous
import jax
import jax.numpy as jnp
from jax import lax
from jax.experimental import pallas as pl
from jax.experimental.pallas import tpu as pltpu
from jax.experimental.pallas import tpu_sc as plsc

N = 100000
E = 1600000
D = 16
H = 32

NSUB = 16
NCORE = 2
NW = NSUB * NCORE
NPAD = 100096
RPS = NPAD // NSUB
EROWS = E // 128
K = 8
F = 48
ZR = RPS // NSUB
CB = 352

WB_A, WB_B = 6256, N - 15 * 6256


def _worker_rows(wid):
    r0 = 390 * wid + jnp.minimum(wid, 20)
    cnt = jnp.where(wid < 20, 391, 390)
    return r0, cnt


def _sc_agg_body(table, src3, dst3, out_acc,
                 acc_sh, idx_s, idx_d, rows_v, z2d, sem):
    c = lax.axis_index("c")
    s = lax.axis_index("s")
    wid = s * NCORE + c

    def _zfill(i, carry):
        z2d[i, :] = jnp.zeros((16,), jnp.float32)
        return carry
    lax.fori_loop(0, ZR, _zfill, 0)
    base = s * RPS
    def _zcopy(k, carry):
        pltpu.sync_copy(z2d, acc_sh.at[pl.ds(base + k * ZR, ZR)])
        return carry
    lax.fori_loop(0, NSUB, _zcopy, 0)
    plsc.subcore_barrier()

    r0, cnt = _worker_rows(wid)
    def _chunk(i, carry):
        rbase = r0 + i * K
        pltpu.sync_copy(src3.at[pl.ds(rbase, K)], idx_s)
        pltpu.sync_copy(dst3.at[pl.ds(rbase, K)], idx_d)
        cps = [
            pltpu.async_copy(table.at[idx_s.at[j]],
                             rows_v.at[pl.ds(j * 128, 128)], sem)
            for j in range(K)
        ]
        for cp in cps:
            cp.wait()
        for j in range(K):
            pltpu.sync_copy(rows_v.at[pl.ds(j * 128, 128)],
                            acc_sh.at[idx_d.at[j]], add=True)
        return carry
    lax.fori_loop(0, F, _chunk, 0)

    def _tail(t, carry):
        r = r0 + t
        pltpu.sync_copy(src3.at[pl.ds(r, 1)], idx_s.at[pl.ds(0, 1)])
        pltpu.sync_copy(dst3.at[pl.ds(r, 1)], idx_d.at[pl.ds(0, 1)])
        pltpu.async_copy(table.at[idx_s.at[0]],
                         rows_v.at[pl.ds(0, 128)], sem).wait()
        pltpu.sync_copy(rows_v.at[pl.ds(0, 128)],
                        acc_sh.at[idx_d.at[0]], add=True)
        return carry
    lax.fori_loop(K * F, cnt, _tail, 0)
    plsc.subcore_barrier()

    @pl.when(s < 15)
    def _():
        pltpu.sync_copy(acc_sh.at[pl.ds(base, WB_A)],
                        out_acc.at[c].at[pl.ds(base, WB_A)])
    @pl.when(s == 15)
    def _():
        pltpu.sync_copy(acc_sh.at[pl.ds(base, WB_B)],
                        out_acc.at[c].at[pl.ds(base, WB_B)])


def _sc_deg_body(dst3, out_degw, deg_sh, idx_d, ones_v, z1, dtile, dbuf, sem):
    c = lax.axis_index("c")
    s = lax.axis_index("s")
    wid = s * NCORE + c

    def _zfill(i, carry):
        z1[pl.ds(i * 16, 16)] = jnp.zeros((16,), jnp.float32)
        return carry
    lax.fori_loop(0, ZR, _zfill, 0)
    base = s * RPS
    pltpu.sync_copy(z1, deg_sh.at[pl.ds(base, RPS)])
    for k in range(8):
        ones_v[pl.ds(k * 16, 16)] = jnp.ones((16,), jnp.float32)
    plsc.subcore_barrier()

    r0, cnt = _worker_rows(wid)
    def _chunk(i, carry):
        rbase = r0 + i * K
        pltpu.sync_copy(dst3.at[pl.ds(rbase, K)], idx_d)
        for j in range(K):
            pltpu.sync_copy(ones_v, deg_sh.at[idx_d.at[j]], add=True)
        return carry
    lax.fori_loop(0, F, _chunk, 0)
    def _tail(t, carry):
        pltpu.sync_copy(dst3.at[pl.ds(r0 + t, 1)], idx_d.at[pl.ds(0, 1)])
        pltpu.sync_copy(ones_v, deg_sh.at[idx_d.at[0]], add=True)
        return carry
    lax.fori_loop(K * F, cnt, _tail, 0)
    plsc.subcore_barrier()

    pltpu.sync_copy(deg_sh.at[pl.ds(base, RPS)], dtile)

    def _fill(roff, ng):
        def _g(g, carry2):
            v = dtile[pl.ds(roff + g * 16, 16)]
            for n in range(16):
                dbuf[g * 16 + n, :] = jnp.full((16,), v[n], jnp.float32)
            return carry2
        lax.fori_loop(0, ng, _g, 0)

    def _bq(q, carry):
        _fill(q * CB, CB // 16)
        pltpu.sync_copy(dbuf, out_degw.at[c].at[pl.ds(base + q * CB, CB)])
        return carry
    lax.fori_loop(0, 17, _bq, 0)
    tb = 17 * CB
    @pl.when(s < 15)
    def _():
        _fill(tb, (WB_A - tb) // 16)
        pltpu.sync_copy(dbuf.at[pl.ds(0, WB_A - tb)],
                        out_degw.at[c].at[pl.ds(base + tb, WB_A - tb)])
    @pl.when(s == 15)
    def _():
        _fill(tb, (WB_B - tb) // 16)
        pltpu.sync_copy(dbuf.at[pl.ds(0, WB_B - tb)],
                        out_degw.at[c].at[pl.ds(base + tb, WB_B - tb)])


_MESH = plsc.VectorSubcoreMesh(core_axis_name="c", subcore_axis_name="s")
_SC_PARAMS = pltpu.CompilerParams(use_tc_tiling_on_sc=False)

_agg = pl.kernel(
    _sc_agg_body,
    compiler_params=_SC_PARAMS,
    out_type=jax.ShapeDtypeStruct((NCORE, N, D), jnp.float32),
    mesh=_MESH,
    scratch_types=[
        pltpu.VMEM_SHARED((NPAD, D), jnp.float32),
        pltpu.VMEM((K, 128), jnp.int32),
        pltpu.VMEM((K, 128), jnp.int32),
        pltpu.VMEM((K * 128, D), jnp.float32),
        pltpu.VMEM((ZR, D), jnp.float32),
        pltpu.SemaphoreType.DMA,
    ],
)

_deg_count = pl.kernel(
    _sc_deg_body,
    compiler_params=_SC_PARAMS,
    out_type=jax.ShapeDtypeStruct((NCORE, N, D), jnp.float32),
    mesh=_MESH,
    scratch_types=[
        pltpu.VMEM_SHARED((NPAD,), jnp.float32),
        pltpu.VMEM((K, 128), jnp.int32),
        pltpu.VMEM((128,), jnp.float32),
        pltpu.VMEM((RPS,), jnp.float32),
        pltpu.VMEM((RPS,), jnp.float32),
        pltpu.VMEM((CB, D), jnp.float32),
        pltpu.SemaphoreType.DMA,
    ],
)

R = 5000
G = N // R


def _dense1_body(p_ref, degw_ref, x_ref, w1l_ref, b1_ref, w1r_ref,
                 w2l_ref, w2r_ref, b2_ref, g_ref, r_ref):
    p = p_ref[0] + p_ref[1]
    d = degw_ref[0] + degw_ref[1]
    mean = p / jnp.clip(d, 1.0)
    h = jnp.maximum(
        mean @ w1l_ref[...] + b1_ref[...] + x_ref[...] @ w1r_ref[...], 0.0)
    g_ref[...] = h @ w2l_ref[...]
    r_ref[...] = h @ w2r_ref[...] + b2_ref[...]


_dense1 = pl.pallas_call(
    _dense1_body,
    grid=(G,),
    in_specs=[
        pl.BlockSpec((NCORE, R, D), lambda i: (0, i, 0)),
        pl.BlockSpec((NCORE, R, D), lambda i: (0, i, 0)),
        pl.BlockSpec((R, D), lambda i: (i, 0)),
        pl.BlockSpec((D, H), lambda i: (0, 0)),
        pl.BlockSpec((1, H), lambda i: (0, 0)),
        pl.BlockSpec((D, H), lambda i: (0, 0)),
        pl.BlockSpec((H, D), lambda i: (0, 0)),
        pl.BlockSpec((H, D), lambda i: (0, 0)),
        pl.BlockSpec((1, D), lambda i: (0, 0)),
    ],
    out_specs=[
        pl.BlockSpec((R, D), lambda i: (i, 0)),
        pl.BlockSpec((R, D), lambda i: (i, 0)),
    ],
    out_shape=[
        jax.ShapeDtypeStruct((N, D), jnp.float32),
        jax.ShapeDtypeStruct((N, D), jnp.float32),
    ],
)


def _dense2_body(p_ref, degw_ref, r_ref, o_ref):
    p = p_ref[0] + p_ref[1]
    d = degw_ref[0] + degw_ref[1]
    o_ref[...] = p / jnp.clip(d, 1.0) + r_ref[...]


_dense2 = pl.pallas_call(
    _dense2_body,
    grid=(G,),
    in_specs=[
        pl.BlockSpec((NCORE, R, D), lambda i: (0, i, 0)),
        pl.BlockSpec((NCORE, R, D), lambda i: (0, i, 0)),
        pl.BlockSpec((R, D), lambda i: (i, 0)),
    ],
    out_specs=pl.BlockSpec((R, D), lambda i: (i, 0)),
    out_shape=jax.ShapeDtypeStruct((N, D), jnp.float32),
)


def kernel(x, edge_index, W1_l, b1_l, W1_r, W2_l, b2_l, W2_r):
    src3 = edge_index[0].reshape(EROWS, 128)
    dst3 = edge_index[1].reshape(EROWS, 128)

    p1 = _agg(x, src3, dst3)
    degw = _deg_count(dst3)
    g, r = _dense1(p1, degw, x, W1_l, b1_l.reshape(1, H), W1_r,
                   W2_l, W2_r, b2_l.reshape(1, D))
    p2 = _agg(g, src3, dst3)
    return _dense2(p2, degw, r)


# speedup vs baseline: 22.9191x; 1.4183x over previous
import jax
import jax.numpy as jnp
from jax import lax
from jax.experimental import pallas as pl
from jax.experimental.pallas import tpu as pltpu
from jax.experimental.pallas import tpu_sc as plsc

N = 100000
E = 1600000
D = 16
H = 32

NSUB = 16
NCORE = 2
NW = NSUB * NCORE
NPAD = 100096
RPS = NPAD // NSUB
WROWS = NPAD // 8
WPS = RPS // 8
EROWS = E // 128
K = 8
F = 48
ZR = RPS // NSUB
CB = 352
WCH = 368


def _worker_rows(wid):
    r0 = 390 * wid + jnp.minimum(wid, 20)
    cnt = jnp.where(wid < 20, 391, 390)
    return r0, cnt


def _sc_agg_body(table, src3, dst3, out_acc,
                 acc_sh, idx_s, idx_d, rows_v, wbuf, sem):
    c = lax.axis_index("c")
    s = lax.axis_index("s")
    wid = s * NCORE + c

    def _zfill(i, carry):
        rows_v[i, :] = jnp.zeros((16,), jnp.float32)
        return carry
    lax.fori_loop(0, K * 128, _zfill, 0)
    base = s * RPS
    def _zcopy(k, carry):
        pltpu.sync_copy(rows_v, acc_sh.at[pl.ds(base + k * K * 128, K * 128)])
        return carry
    lax.fori_loop(0, RPS // (K * 128), _zcopy, 0)
    zt = (RPS // (K * 128)) * K * 128
    pltpu.sync_copy(rows_v.at[pl.ds(0, RPS - zt)],
                    acc_sh.at[pl.ds(base + zt, RPS - zt)])
    plsc.subcore_barrier()

    r0, cnt = _worker_rows(wid)
    def _chunk(i, carry):
        rbase = r0 + i * K
        pltpu.sync_copy(src3.at[pl.ds(rbase, K)], idx_s)
        pltpu.sync_copy(dst3.at[pl.ds(rbase, K)], idx_d)
        cps = [
            pltpu.async_copy(table.at[idx_s.at[j]],
                             rows_v.at[pl.ds(j * 128, 128)], sem)
            for j in range(K)
        ]
        for cp in cps:
            cp.wait()
        for j in range(K):
            pltpu.sync_copy(rows_v.at[pl.ds(j * 128, 128)],
                            acc_sh.at[idx_d.at[j]], add=True)
        return carry
    lax.fori_loop(0, F, _chunk, 0)

    def _tail(t, carry):
        r = r0 + t
        pltpu.sync_copy(src3.at[pl.ds(r, 1)], idx_s.at[pl.ds(0, 1)])
        pltpu.sync_copy(dst3.at[pl.ds(r, 1)], idx_d.at[pl.ds(0, 1)])
        pltpu.async_copy(table.at[idx_s.at[0]],
                         rows_v.at[pl.ds(0, 128)], sem).wait()
        pltpu.sync_copy(rows_v.at[pl.ds(0, 128)],
                        acc_sh.at[idx_d.at[0]], add=True)
        return carry
    lax.fori_loop(K * F, cnt, _tail, 0)
    plsc.subcore_barrier()

    def _wb(h, carry):
        pltpu.sync_copy(acc_sh.at[pl.ds(base + h * WCH, WCH)],
                        rows_v.at[pl.ds(0, WCH)])
        def _rep(w, carry2):
            for n in range(8):
                wbuf[w, pl.ds(16 * n, 16)] = rows_v[8 * w + n, :]
            return carry2
        lax.fori_loop(0, WCH // 8, _rep, 0)
        pltpu.sync_copy(wbuf,
                        out_acc.at[c].at[pl.ds(s * WPS + h * (WCH // 8),
                                               WCH // 8)])
        return carry
    lax.fori_loop(0, RPS // WCH, _wb, 0)


def _sc_deg_body(dst3, out_degw, deg_sh, idx_d, ones_v, z1, dtile, dbuf, sem):
    c = lax.axis_index("c")
    s = lax.axis_index("s")
    wid = s * NCORE + c

    def _zfill(i, carry):
        z1[pl.ds(i * 16, 16)] = jnp.zeros((16,), jnp.float32)
        return carry
    lax.fori_loop(0, ZR, _zfill, 0)
    base = s * RPS
    pltpu.sync_copy(z1, deg_sh.at[pl.ds(base, RPS)])
    for k in range(8):
        ones_v[pl.ds(k * 16, 16)] = jnp.ones((16,), jnp.float32)
    plsc.subcore_barrier()

    r0, cnt = _worker_rows(wid)
    def _chunk(i, carry):
        rbase = r0 + i * K
        pltpu.sync_copy(dst3.at[pl.ds(rbase, K)], idx_d)
        for j in range(K):
            pltpu.sync_copy(ones_v, deg_sh.at[idx_d.at[j]], add=True)
        return carry
    lax.fori_loop(0, F, _chunk, 0)
    def _tail(t, carry):
        pltpu.sync_copy(dst3.at[pl.ds(r0 + t, 1)], idx_d.at[pl.ds(0, 1)])
        pltpu.sync_copy(ones_v, deg_sh.at[idx_d.at[0]], add=True)
        return carry
    lax.fori_loop(K * F, cnt, _tail, 0)
    plsc.subcore_barrier()

    pltpu.sync_copy(deg_sh.at[pl.ds(base, RPS)], dtile)

    def _fill(roff, ng):
        def _g(g, carry2):
            v = dtile[pl.ds(roff + g * 16, 16)]
            for n in range(8):
                dbuf[2 * g, pl.ds(16 * n, 16)] = jnp.full((16,), v[n],
                                                          jnp.float32)
            for n in range(8):
                dbuf[2 * g + 1, pl.ds(16 * n, 16)] = jnp.full(
                    (16,), v[8 + n], jnp.float32)
            return carry2
        lax.fori_loop(0, ng, _g, 0)

    def _bq(q, carry):
        _fill(q * CB, CB // 16)
        pltpu.sync_copy(dbuf, out_degw.at[c].at[pl.ds(s * WPS + q * (CB // 8),
                                                      CB // 8)])
        return carry
    lax.fori_loop(0, 17, _bq, 0)
    tb = 17 * CB
    _fill(tb, (RPS - tb) // 16)
    pltpu.sync_copy(dbuf.at[pl.ds(0, (RPS - tb) // 8)],
                    out_degw.at[c].at[pl.ds(s * WPS + tb // 8,
                                            (RPS - tb) // 8)])


_MESH = plsc.VectorSubcoreMesh(core_axis_name="c", subcore_axis_name="s")
_SC_PARAMS = pltpu.CompilerParams(use_tc_tiling_on_sc=False)

_agg = pl.kernel(
    _sc_agg_body,
    compiler_params=_SC_PARAMS,
    out_type=jax.ShapeDtypeStruct((NCORE, WROWS, 128), jnp.float32),
    mesh=_MESH,
    scratch_types=[
        pltpu.VMEM_SHARED((NPAD, D), jnp.float32),
        pltpu.VMEM((K, 128), jnp.int32),
        pltpu.VMEM((K, 128), jnp.int32),
        pltpu.VMEM((K * 128, D), jnp.float32),
        pltpu.VMEM((WCH // 8, 128), jnp.float32),
        pltpu.SemaphoreType.DMA,
    ],
)

_deg_count = pl.kernel(
    _sc_deg_body,
    compiler_params=_SC_PARAMS,
    out_type=jax.ShapeDtypeStruct((NCORE, WROWS, 128), jnp.float32),
    mesh=_MESH,
    scratch_types=[
        pltpu.VMEM_SHARED((NPAD,), jnp.float32),
        pltpu.VMEM((K, 128), jnp.int32),
        pltpu.VMEM((128,), jnp.float32),
        pltpu.VMEM((RPS,), jnp.float32),
        pltpu.VMEM((RPS,), jnp.float32),
        pltpu.VMEM((CB // 8, 128), jnp.float32),
        pltpu.SemaphoreType.DMA,
    ],
)

RB = 3128
G = WROWS // RB


def _dense1_body(p_ref, degw_ref, x_ref, w1l_ref, b1_ref, w1r_ref,
                 w2l_ref, w2r_ref, b2_ref, g_ref, r_ref):
    p = p_ref[0] + p_ref[1]
    d = degw_ref[0] + degw_ref[1]
    mean = p / jnp.clip(d, 1.0)
    h = jnp.maximum(
        mean @ w1l_ref[...] + b1_ref[...] + x_ref[...] @ w1r_ref[...], 0.0)
    g_ref[...] = h @ w2l_ref[...]
    r_ref[...] = h @ w2r_ref[...] + b2_ref[...]


_dense1 = pl.pallas_call(
    _dense1_body,
    grid=(G,),
    in_specs=[
        pl.BlockSpec((NCORE, RB, 128), lambda i: (0, i, 0)),
        pl.BlockSpec((NCORE, RB, 128), lambda i: (0, i, 0)),
        pl.BlockSpec((RB, 128), lambda i: (i, 0)),
        pl.BlockSpec((128, 8 * H), lambda i: (0, 0)),
        pl.BlockSpec((1, 8 * H), lambda i: (0, 0)),
        pl.BlockSpec((128, 8 * H), lambda i: (0, 0)),
        pl.BlockSpec((8 * H, 128), lambda i: (0, 0)),
        pl.BlockSpec((8 * H, 128), lambda i: (0, 0)),
        pl.BlockSpec((1, 128), lambda i: (0, 0)),
    ],
    out_specs=[
        pl.BlockSpec((RB, 128), lambda i: (i, 0)),
        pl.BlockSpec((RB, 128), lambda i: (i, 0)),
    ],
    out_shape=[
        jax.ShapeDtypeStruct((WROWS, 128), jnp.float32),
        jax.ShapeDtypeStruct((WROWS, 128), jnp.float32),
    ],
)


def _dense2_body(p_ref, degw_ref, r_ref, o_ref):
    p = p_ref[0] + p_ref[1]
    d = degw_ref[0] + degw_ref[1]
    o_ref[...] = p / jnp.clip(d, 1.0) + r_ref[...]


_dense2 = pl.pallas_call(
    _dense2_body,
    grid=(G,),
    in_specs=[
        pl.BlockSpec((NCORE, RB, 128), lambda i: (0, i, 0)),
        pl.BlockSpec((NCORE, RB, 128), lambda i: (0, i, 0)),
        pl.BlockSpec((RB, 128), lambda i: (i, 0)),
    ],
    out_specs=pl.BlockSpec((RB, 128), lambda i: (i, 0)),
    out_shape=jax.ShapeDtypeStruct((WROWS, 128), jnp.float32),
)


def kernel(x, edge_index, W1_l, b1_l, W1_r, W2_l, b2_l, W2_r):
    f32 = jnp.float32
    src3 = edge_index[0].reshape(EROWS, 128)
    dst3 = edge_index[1].reshape(EROWS, 128)
    xw = jnp.pad(x.reshape(E // 128, 128), ((0, WROWS - E // 128), (0, 0)))

    eye8 = jnp.eye(8, dtype=f32)
    w1l_w = jnp.kron(eye8, W1_l)
    w1r_w = jnp.kron(eye8, W1_r)
    w2l_w = jnp.kron(eye8, W2_l)
    w2r_w = jnp.kron(eye8, W2_r)
    b1_w = jnp.tile(b1_l, 8).reshape(1, 8 * H)
    b2_w = jnp.tile(b2_l, 8).reshape(1, 128)

    p1 = _agg(x, src3, dst3)
    degw = _deg_count(dst3)
    gw, rw = _dense1(p1, degw, xw, w1l_w, b1_w, w1r_w, w2l_w, w2r_w, b2_w)
    p2 = _agg(gw.reshape(NPAD, D), src3, dst3)
    outw = _dense2(p2, degw, rw)
    return outw[:E // 128].reshape(N, D)


# speedup vs baseline: 28.8301x; 1.2579x over previous
import jax
import jax.numpy as jnp
from jax import lax
from jax.experimental import pallas as pl
from jax.experimental.pallas import tpu as pltpu
from jax.experimental.pallas import tpu_sc as plsc

N = 100000
E = 1600000
D = 16
H = 32

NSUB = 16
NCORE = 2
NW = NSUB * NCORE
NPAD = 100096
RPS = NPAD // NSUB
WROWS = NPAD // 8
WPS = RPS // 8
EROWS = E // 128
K = 5
F = 78
ZR = RPS // NSUB
CB = 352
WCH = 368


def _worker_rows(wid):
    r0 = 390 * wid + jnp.minimum(wid, 20)
    cnt = jnp.where(wid < 20, 391, 390)
    return r0, cnt


def _sc_agg_body(table, src3, dst3, out_acc,
                 acc_sh, idx_sa, idx_da, idx_sb, idx_db,
                 rows_a, rows_b, wbuf, sg, ss):
    c = lax.axis_index("c")
    s = lax.axis_index("s")
    wid = s * NCORE + c

    def _zfill(i, carry):
        rows_a[i, :] = jnp.zeros((16,), jnp.float32)
        return carry
    lax.fori_loop(0, K * 128, _zfill, 0)
    base = s * RPS
    def _zcopy(k, carry):
        pltpu.sync_copy(rows_a, acc_sh.at[pl.ds(base + k * K * 128, K * 128)])
        return carry
    lax.fori_loop(0, RPS // (K * 128), _zcopy, 0)
    zt = (RPS // (K * 128)) * K * 128
    pltpu.sync_copy(rows_a.at[pl.ds(0, RPS - zt)],
                    acc_sh.at[pl.ds(base + zt, RPS - zt)])
    plsc.subcore_barrier()

    r0, cnt = _worker_rows(wid)

    def _fire(i, idx_s, idx_d, rows):
        rb = r0 + i * K
        pltpu.sync_copy(src3.at[pl.ds(rb, K)], idx_s)
        pltpu.sync_copy(dst3.at[pl.ds(rb, K)], idx_d)
        for j in range(K):
            pltpu.async_copy(table.at[idx_s.at[j]],
                             rows.at[pl.ds(j * 128, 128)], sg)

    def _drain(sem, nstreams):
        for j in range(nstreams):
            pltpu.make_async_copy(table.at[pl.ds(0, 128)],
                                  rows_a.at[pl.ds(0, 128)], sem).wait()

    def _scat(idx_d, rows):
        for j in range(K):
            pltpu.async_copy(rows.at[pl.ds(j * 128, 128)],
                             acc_sh.at[idx_d.at[j]], ss, add=True)

    _fire(0, idx_sa, idx_da, rows_a)
    def _pair(t, carry):
        @pl.when(t > 0)
        def _():
            _drain(ss, K)
        _fire(2 * t + 1, idx_sb, idx_db, rows_b)
        _drain(sg, K)
        _scat(idx_da, rows_a)
        @pl.when(t < F // 2 - 1)
        def _():
            _drain(ss, K)
            _fire(2 * t + 2, idx_sa, idx_da, rows_a)
        _drain(sg, K)
        _scat(idx_db, rows_b)
        return carry
    lax.fori_loop(0, F // 2, _pair, 0)
    _drain(ss, 2 * K)

    def _tail(t, carry):
        r = r0 + t
        pltpu.sync_copy(src3.at[pl.ds(r, 1)], idx_sa.at[pl.ds(0, 1)])
        pltpu.sync_copy(dst3.at[pl.ds(r, 1)], idx_da.at[pl.ds(0, 1)])
        pltpu.async_copy(table.at[idx_sa.at[0]],
                         rows_a.at[pl.ds(0, 128)], sg).wait()
        pltpu.sync_copy(rows_a.at[pl.ds(0, 128)],
                        acc_sh.at[idx_da.at[0]], add=True)
        return carry
    lax.fori_loop(K * F, cnt, _tail, 0)
    plsc.subcore_barrier()

    def _wb(h, carry):
        pltpu.sync_copy(acc_sh.at[pl.ds(base + h * WCH, WCH)],
                        rows_a.at[pl.ds(0, WCH)])
        def _rep(w, carry2):
            for n in range(8):
                wbuf[w, pl.ds(16 * n, 16)] = rows_a[8 * w + n, :]
            return carry2
        lax.fori_loop(0, WCH // 8, _rep, 0)
        pltpu.sync_copy(wbuf,
                        out_acc.at[c].at[pl.ds(s * WPS + h * (WCH // 8),
                                               WCH // 8)])
        return carry
    lax.fori_loop(0, RPS // WCH, _wb, 0)


def _sc_deg_body(dst3, out_degw, deg_sh, idx_d, ones_v, z1, dtile, dbuf, sem):
    c = lax.axis_index("c")
    s = lax.axis_index("s")
    wid = s * NCORE + c

    def _zfill(i, carry):
        z1[pl.ds(i * 16, 16)] = jnp.zeros((16,), jnp.float32)
        return carry
    lax.fori_loop(0, ZR, _zfill, 0)
    base = s * RPS
    pltpu.sync_copy(z1, deg_sh.at[pl.ds(base, RPS)])
    for k in range(8):
        ones_v[pl.ds(k * 16, 16)] = jnp.ones((16,), jnp.float32)
    plsc.subcore_barrier()

    r0, cnt = _worker_rows(wid)
    def _chunk(i, carry):
        rbase = r0 + i * K
        pltpu.sync_copy(dst3.at[pl.ds(rbase, K)], idx_d)
        for j in range(K):
            pltpu.sync_copy(ones_v, deg_sh.at[idx_d.at[j]], add=True)
        return carry
    lax.fori_loop(0, F, _chunk, 0)
    def _tail(t, carry):
        pltpu.sync_copy(dst3.at[pl.ds(r0 + t, 1)], idx_d.at[pl.ds(0, 1)])
        pltpu.sync_copy(ones_v, deg_sh.at[idx_d.at[0]], add=True)
        return carry
    lax.fori_loop(K * F, cnt, _tail, 0)
    plsc.subcore_barrier()

    pltpu.sync_copy(deg_sh.at[pl.ds(base, RPS)], dtile)

    def _fill(roff, ng):
        def _g(g, carry2):
            v = dtile[pl.ds(roff + g * 16, 16)]
            for n in range(8):
                dbuf[2 * g, pl.ds(16 * n, 16)] = jnp.full((16,), v[n],
                                                          jnp.float32)
            for n in range(8):
                dbuf[2 * g + 1, pl.ds(16 * n, 16)] = jnp.full(
                    (16,), v[8 + n], jnp.float32)
            return carry2
        lax.fori_loop(0, ng, _g, 0)

    def _bq(q, carry):
        _fill(q * CB, CB // 16)
        pltpu.sync_copy(dbuf, out_degw.at[c].at[pl.ds(s * WPS + q * (CB // 8),
                                                      CB // 8)])
        return carry
    lax.fori_loop(0, 17, _bq, 0)
    tb = 17 * CB
    _fill(tb, (RPS - tb) // 16)
    pltpu.sync_copy(dbuf.at[pl.ds(0, (RPS - tb) // 8)],
                    out_degw.at[c].at[pl.ds(s * WPS + tb // 8,
                                            (RPS - tb) // 8)])


_MESH = plsc.VectorSubcoreMesh(core_axis_name="c", subcore_axis_name="s")
_SC_PARAMS = pltpu.CompilerParams(use_tc_tiling_on_sc=False)

_agg = pl.kernel(
    _sc_agg_body,
    compiler_params=_SC_PARAMS,
    out_type=jax.ShapeDtypeStruct((NCORE, WROWS, 128), jnp.float32),
    mesh=_MESH,
    scratch_types=[
        pltpu.VMEM_SHARED((NPAD, D), jnp.float32),
        pltpu.VMEM((K, 128), jnp.int32),
        pltpu.VMEM((K, 128), jnp.int32),
        pltpu.VMEM((K, 128), jnp.int32),
        pltpu.VMEM((K, 128), jnp.int32),
        pltpu.VMEM((K * 128, D), jnp.float32),
        pltpu.VMEM((K * 128, D), jnp.float32),
        pltpu.VMEM((WCH // 8, 128), jnp.float32),
        pltpu.SemaphoreType.DMA,
        pltpu.SemaphoreType.DMA,
    ],
)

_deg_count = pl.kernel(
    _sc_deg_body,
    compiler_params=_SC_PARAMS,
    out_type=jax.ShapeDtypeStruct((NCORE, WROWS, 128), jnp.float32),
    mesh=_MESH,
    scratch_types=[
        pltpu.VMEM_SHARED((NPAD,), jnp.float32),
        pltpu.VMEM((K, 128), jnp.int32),
        pltpu.VMEM((128,), jnp.float32),
        pltpu.VMEM((RPS,), jnp.float32),
        pltpu.VMEM((RPS,), jnp.float32),
        pltpu.VMEM((CB // 8, 128), jnp.float32),
        pltpu.SemaphoreType.DMA,
    ],
)

RB = 3128
G = WROWS // RB


def _dense1_body(p_ref, degw_ref, x_ref, w1l_ref, b1_ref, w1r_ref,
                 w2l_ref, w2r_ref, b2_ref, g_ref, r_ref):
    p = p_ref[0] + p_ref[1]
    d = degw_ref[0] + degw_ref[1]
    mean = p / jnp.clip(d, 1.0)
    h = jnp.maximum(
        mean @ w1l_ref[...] + b1_ref[...] + x_ref[...] @ w1r_ref[...], 0.0)
    g_ref[...] = h @ w2l_ref[...]
    r_ref[...] = h @ w2r_ref[...] + b2_ref[...]


_dense1 = pl.pallas_call(
    _dense1_body,
    grid=(G,),
    in_specs=[
        pl.BlockSpec((NCORE, RB, 128), lambda i: (0, i, 0)),
        pl.BlockSpec((NCORE, RB, 128), lambda i: (0, i, 0)),
        pl.BlockSpec((RB, 128), lambda i: (i, 0)),
        pl.BlockSpec((128, 8 * H), lambda i: (0, 0)),
        pl.BlockSpec((1, 8 * H), lambda i: (0, 0)),
        pl.BlockSpec((128, 8 * H), lambda i: (0, 0)),
        pl.BlockSpec((8 * H, 128), lambda i: (0, 0)),
        pl.BlockSpec((8 * H, 128), lambda i: (0, 0)),
        pl.BlockSpec((1, 128), lambda i: (0, 0)),
    ],
    out_specs=[
        pl.BlockSpec((RB, 128), lambda i: (i, 0)),
        pl.BlockSpec((RB, 128), lambda i: (i, 0)),
    ],
    out_shape=[
        jax.ShapeDtypeStruct((WROWS, 128), jnp.float32),
        jax.ShapeDtypeStruct((WROWS, 128), jnp.float32),
    ],
)


def _dense2_body(p_ref, degw_ref, r_ref, o_ref):
    p = p_ref[0] + p_ref[1]
    d = degw_ref[0] + degw_ref[1]
    o_ref[...] = p / jnp.clip(d, 1.0) + r_ref[...]


_dense2 = pl.pallas_call(
    _dense2_body,
    grid=(G,),
    in_specs=[
        pl.BlockSpec((NCORE, RB, 128), lambda i: (0, i, 0)),
        pl.BlockSpec((NCORE, RB, 128), lambda i: (0, i, 0)),
        pl.BlockSpec((RB, 128), lambda i: (i, 0)),
    ],
    out_specs=pl.BlockSpec((RB, 128), lambda i: (i, 0)),
    out_shape=jax.ShapeDtypeStruct((WROWS, 128), jnp.float32),
)


def kernel(x, edge_index, W1_l, b1_l, W1_r, W2_l, b2_l, W2_r):
    f32 = jnp.float32
    src3 = edge_index[0].reshape(EROWS, 128)
    dst3 = edge_index[1].reshape(EROWS, 128)
    xw = jnp.pad(x.reshape(E // 128, 128), ((0, WROWS - E // 128), (0, 0)))

    eye8 = jnp.eye(8, dtype=f32)
    w1l_w = jnp.kron(eye8, W1_l)
    w1r_w = jnp.kron(eye8, W1_r)
    w2l_w = jnp.kron(eye8, W2_l)
    w2r_w = jnp.kron(eye8, W2_r)
    b1_w = jnp.tile(b1_l, 8).reshape(1, 8 * H)
    b2_w = jnp.tile(b2_l, 8).reshape(1, 128)

    p1 = _agg(x, src3, dst3)
    degw = _deg_count(dst3)
    gw, rw = _dense1(p1, degw, xw, w1l_w, b1_w, w1r_w, w2l_w, w2r_w, b2_w)
    p2 = _agg(gw.reshape(NPAD, D), src3, dst3)
    outw = _dense2(p2, degw, rw)
    return outw[:E // 128].reshape(N, D)


# speedup vs baseline: 32.6313x; 1.1318x over previous
import jax
import jax.numpy as jnp
from jax import lax
from jax.experimental import pallas as pl
from jax.experimental.pallas import tpu as pltpu
from jax.experimental.pallas import tpu_sc as plsc

N = 100000
E = 1600000
D = 16
H = 32

NSUB = 16
NCORE = 2
NW = NSUB * NCORE
NPAD = 100096
RPS = NPAD // NSUB
WROWS = NPAD // 8
WPS = RPS // 8
EROWS = E // 128
K = 5
F = 78
ZR = RPS // NSUB
CB = 352
WCH = 368


def _worker_rows(wid):
    r0 = 390 * wid + jnp.minimum(wid, 20)
    cnt = jnp.where(wid < 20, 391, 390)
    return r0, cnt


def _sc_agg_body(table, edge3, out_acc,
                 acc_sh, idx_sa, idx_da, idx_sb, idx_db,
                 rows_a, rows_b, wbuf, sg, ss):
    c = lax.axis_index("c")
    s = lax.axis_index("s")
    wid = s * NCORE + c

    def _zfill(i, carry):
        rows_a[i, :] = jnp.zeros((16,), jnp.float32)
        return carry
    lax.fori_loop(0, K * 128, _zfill, 0)
    base = s * RPS
    def _zcopy(k, carry):
        pltpu.sync_copy(rows_a, acc_sh.at[pl.ds(base + k * K * 128, K * 128)])
        return carry
    lax.fori_loop(0, RPS // (K * 128), _zcopy, 0)
    zt = (RPS // (K * 128)) * K * 128
    pltpu.sync_copy(rows_a.at[pl.ds(0, RPS - zt)],
                    acc_sh.at[pl.ds(base + zt, RPS - zt)])
    plsc.subcore_barrier()

    r0, cnt = _worker_rows(wid)

    def _fire(i, idx_s, idx_d, rows):
        rb = r0 + i * K
        pltpu.sync_copy(edge3.at[0].at[pl.ds(rb, K)], idx_s)
        pltpu.sync_copy(edge3.at[1].at[pl.ds(rb, K)], idx_d)
        for j in range(K):
            pltpu.async_copy(table.at[idx_s.at[j]],
                             rows.at[pl.ds(j * 128, 128)], sg)

    def _drain(sem, nstreams):
        for j in range(nstreams):
            pltpu.make_async_copy(table.at[pl.ds(0, 128)],
                                  rows_a.at[pl.ds(0, 128)], sem).wait()

    def _scat(idx_d, rows):
        for j in range(K):
            pltpu.async_copy(rows.at[pl.ds(j * 128, 128)],
                             acc_sh.at[idx_d.at[j]], ss, add=True)

    _fire(0, idx_sa, idx_da, rows_a)
    def _pair(t, carry):
        @pl.when(t > 0)
        def _():
            _drain(ss, K)
        _fire(2 * t + 1, idx_sb, idx_db, rows_b)
        _drain(sg, K)
        _scat(idx_da, rows_a)
        @pl.when(t < F // 2 - 1)
        def _():
            _drain(ss, K)
            _fire(2 * t + 2, idx_sa, idx_da, rows_a)
        _drain(sg, K)
        _scat(idx_db, rows_b)
        return carry
    lax.fori_loop(0, F // 2, _pair, 0)
    _drain(ss, 2 * K)

    def _tail(t, carry):
        r = r0 + t
        pltpu.sync_copy(edge3.at[0].at[pl.ds(r, 1)], idx_sa.at[pl.ds(0, 1)])
        pltpu.sync_copy(edge3.at[1].at[pl.ds(r, 1)], idx_da.at[pl.ds(0, 1)])
        pltpu.async_copy(table.at[idx_sa.at[0]],
                         rows_a.at[pl.ds(0, 128)], sg).wait()
        pltpu.sync_copy(rows_a.at[pl.ds(0, 128)],
                        acc_sh.at[idx_da.at[0]], add=True)
        return carry
    lax.fori_loop(K * F, cnt, _tail, 0)
    plsc.subcore_barrier()

    def _wb(h, carry):
        pltpu.sync_copy(acc_sh.at[pl.ds(base + h * WCH, WCH)],
                        rows_a.at[pl.ds(0, WCH)])
        def _rep(w, carry2):
            for n in range(8):
                wbuf[w, pl.ds(16 * n, 16)] = rows_a[8 * w + n, :]
            return carry2
        lax.fori_loop(0, WCH // 8, _rep, 0)
        pltpu.sync_copy(wbuf,
                        out_acc.at[c].at[pl.ds(s * WPS + h * (WCH // 8),
                                               WCH // 8)])
        return carry
    lax.fori_loop(0, RPS // WCH, _wb, 0)


def _sc_deg_body(edge3, out_degw, deg_sh, idx_a, idx_b, ones_v, z1,
                 dtile, dbuf, ss):
    c = lax.axis_index("c")
    s = lax.axis_index("s")
    wid = s * NCORE + c

    def _zfill(i, carry):
        z1[pl.ds(i * 16, 16)] = jnp.zeros((16,), jnp.float32)
        return carry
    lax.fori_loop(0, ZR, _zfill, 0)
    base = s * RPS
    pltpu.sync_copy(z1, deg_sh.at[pl.ds(base, RPS)])
    for k in range(8):
        ones_v[pl.ds(k * 16, 16)] = jnp.ones((16,), jnp.float32)
    plsc.subcore_barrier()

    r0, cnt = _worker_rows(wid)

    def _load(i, idx):
        pltpu.sync_copy(edge3.at[1].at[pl.ds(r0 + i * K, K)], idx)

    def _scat(idx):
        for j in range(K):
            pltpu.async_copy(ones_v, deg_sh.at[idx.at[j]], ss, add=True)

    def _drain(n):
        for j in range(n):
            pltpu.make_async_copy(out_degw.at[c].at[0],
                                  ones_v, ss).wait()

    _load(0, idx_a)
    def _pair(t, carry):
        _scat(idx_a)
        @pl.when(t > 0)
        def _():
            _drain(K)
        _load(2 * t + 1, idx_b)
        _scat(idx_b)
        _drain(K)
        @pl.when(t < F // 2 - 1)
        def _():
            _load(2 * t + 2, idx_a)
        return carry
    lax.fori_loop(0, F // 2, _pair, 0)
    _drain(K)

    def _tail(t, carry):
        pltpu.sync_copy(edge3.at[1].at[pl.ds(r0 + t, 1)],
                        idx_a.at[pl.ds(0, 1)])
        pltpu.sync_copy(ones_v, deg_sh.at[idx_a.at[0]], add=True)
        return carry
    lax.fori_loop(K * F, cnt, _tail, 0)
    plsc.subcore_barrier()

    pltpu.sync_copy(deg_sh.at[pl.ds(base, RPS)], dtile)

    def _fill(roff, ng):
        def _g(g, carry2):
            v = dtile[pl.ds(roff + g * 16, 16)]
            for n in range(8):
                dbuf[2 * g, pl.ds(16 * n, 16)] = jnp.full((16,), v[n],
                                                          jnp.float32)
            for n in range(8):
                dbuf[2 * g + 1, pl.ds(16 * n, 16)] = jnp.full(
                    (16,), v[8 + n], jnp.float32)
            return carry2
        lax.fori_loop(0, ng, _g, 0)

    def _bq(q, carry):
        _fill(q * CB, CB // 16)
        pltpu.sync_copy(dbuf, out_degw.at[c].at[pl.ds(s * WPS + q * (CB // 8),
                                                      CB // 8)])
        return carry
    lax.fori_loop(0, 17, _bq, 0)
    tb = 17 * CB
    _fill(tb, (RPS - tb) // 16)
    pltpu.sync_copy(dbuf.at[pl.ds(0, (RPS - tb) // 8)],
                    out_degw.at[c].at[pl.ds(s * WPS + tb // 8,
                                            (RPS - tb) // 8)])


_MESH = plsc.VectorSubcoreMesh(core_axis_name="c", subcore_axis_name="s")
_SC_PARAMS = pltpu.CompilerParams(use_tc_tiling_on_sc=False)

_agg = pl.kernel(
    _sc_agg_body,
    compiler_params=_SC_PARAMS,
    out_type=jax.ShapeDtypeStruct((NCORE, WROWS, 128), jnp.float32),
    mesh=_MESH,
    scratch_types=[
        pltpu.VMEM_SHARED((NPAD, D), jnp.float32),
        pltpu.VMEM((K, 128), jnp.int32),
        pltpu.VMEM((K, 128), jnp.int32),
        pltpu.VMEM((K, 128), jnp.int32),
        pltpu.VMEM((K, 128), jnp.int32),
        pltpu.VMEM((K * 128, D), jnp.float32),
        pltpu.VMEM((K * 128, D), jnp.float32),
        pltpu.VMEM((WCH // 8, 128), jnp.float32),
        pltpu.SemaphoreType.DMA,
        pltpu.SemaphoreType.DMA,
    ],
)

_deg_count = pl.kernel(
    _sc_deg_body,
    compiler_params=_SC_PARAMS,
    out_type=jax.ShapeDtypeStruct((NCORE, WROWS, 128), jnp.float32),
    mesh=_MESH,
    scratch_types=[
        pltpu.VMEM_SHARED((NPAD,), jnp.float32),
        pltpu.VMEM((K, 128), jnp.int32),
        pltpu.VMEM((K, 128), jnp.int32),
        pltpu.VMEM((128,), jnp.float32),
        pltpu.VMEM((RPS,), jnp.float32),
        pltpu.VMEM((RPS,), jnp.float32),
        pltpu.VMEM((CB // 8, 128), jnp.float32),
        pltpu.SemaphoreType.DMA,
    ],
)

RB = 3128
G = WROWS // RB


def _dense1_body(p_ref, degw_ref, x_ref, w1l_ref, b1_ref, w1r_ref,
                 w2l_ref, w2r_ref, b2_ref, g_ref, r_ref):
    p = p_ref[0] + p_ref[1]
    d = degw_ref[0] + degw_ref[1]
    mean = p / jnp.clip(d, 1.0)
    h = jnp.maximum(
        mean @ w1l_ref[...] + b1_ref[...] + x_ref[...] @ w1r_ref[...], 0.0)
    g_ref[...] = h @ w2l_ref[...]
    r_ref[...] = h @ w2r_ref[...] + b2_ref[...]


_dense1 = pl.pallas_call(
    _dense1_body,
    grid=(G,),
    in_specs=[
        pl.BlockSpec((NCORE, RB, 128), lambda i: (0, i, 0)),
        pl.BlockSpec((NCORE, RB, 128), lambda i: (0, i, 0)),
        pl.BlockSpec((RB, 128), lambda i: (i, 0)),
        pl.BlockSpec((128, 8 * H), lambda i: (0, 0)),
        pl.BlockSpec((1, 8 * H), lambda i: (0, 0)),
        pl.BlockSpec((128, 8 * H), lambda i: (0, 0)),
        pl.BlockSpec((8 * H, 128), lambda i: (0, 0)),
        pl.BlockSpec((8 * H, 128), lambda i: (0, 0)),
        pl.BlockSpec((1, 128), lambda i: (0, 0)),
    ],
    out_specs=[
        pl.BlockSpec((RB, 128), lambda i: (i, 0)),
        pl.BlockSpec((RB, 128), lambda i: (i, 0)),
    ],
    out_shape=[
        jax.ShapeDtypeStruct((WROWS, 128), jnp.float32),
        jax.ShapeDtypeStruct((WROWS, 128), jnp.float32),
    ],
)


def _dense2_body(p_ref, degw_ref, r_ref, o_ref):
    p = p_ref[0] + p_ref[1]
    d = degw_ref[0] + degw_ref[1]
    o_ref[...] = p / jnp.clip(d, 1.0) + r_ref[...]


_dense2 = pl.pallas_call(
    _dense2_body,
    grid=(G,),
    in_specs=[
        pl.BlockSpec((NCORE, RB, 128), lambda i: (0, i, 0)),
        pl.BlockSpec((NCORE, RB, 128), lambda i: (0, i, 0)),
        pl.BlockSpec((RB, 128), lambda i: (i, 0)),
    ],
    out_specs=pl.BlockSpec((RB, 128), lambda i: (i, 0)),
    out_shape=jax.ShapeDtypeStruct((WROWS, 128), jnp.float32),
)


def kernel(x, edge_index, W1_l, b1_l, W1_r, W2_l, b2_l, W2_r):
    f32 = jnp.float32
    edge3 = edge_index.reshape(2, EROWS, 128)
    xw = jnp.pad(x.reshape(E // 128, 128), ((0, WROWS - E // 128), (0, 0)))

    eye8 = jnp.eye(8, dtype=f32)
    w1l_w = jnp.kron(eye8, W1_l)
    w1r_w = jnp.kron(eye8, W1_r)
    w2l_w = jnp.kron(eye8, W2_l)
    w2r_w = jnp.kron(eye8, W2_r)
    b1_w = jnp.tile(b1_l, 8).reshape(1, 8 * H)
    b2_w = jnp.tile(b2_l, 8).reshape(1, 128)

    p1 = _agg(x, edge3)
    degw = _deg_count(edge3)
    gw, rw = _dense1(p1, degw, xw, w1l_w, b1_w, w1r_w, w2l_w, w2r_w, b2_w)
    p2 = _agg(gw.reshape(NPAD, D), edge3)
    outw = _dense2(p2, degw, rw)
    return outw[:E // 128].reshape(N, D)


# speedup vs baseline: 34.6697x; 1.0625x over previous
import jax
import jax.numpy as jnp
from jax import lax
from jax.experimental import pallas as pl
from jax.experimental.pallas import tpu as pltpu
from jax.experimental.pallas import tpu_sc as plsc

N = 100000
E = 1600000
D = 16
H = 32

NSUB = 16
NCORE = 2
NW = NSUB * NCORE
NPAD = 100096
RPS = NPAD // NSUB
WROWS = NPAD // 8
WPS = RPS // 8
EROWS = E // 128
K = 5
F = 78
ZR = RPS // NSUB
CB = 352
WCH = 368


def _worker_rows(wid):
    r0 = 390 * wid + jnp.minimum(wid, 20)
    cnt = jnp.where(wid < 20, 391, 390)
    return r0, cnt


def _sc_agg_body(table, edge3, out_acc,
                 acc_sh, idx_sa, idx_da, idx_sb, idx_db,
                 rows_a, rows_b, wbuf, sg, ss, si):
    c = lax.axis_index("c")
    s = lax.axis_index("s")
    wid = s * NCORE + c

    def _zfill(i, carry):
        rows_a[i, :] = jnp.zeros((16,), jnp.float32)
        return carry
    lax.fori_loop(0, K * 128, _zfill, 0)
    base = s * RPS
    def _zcopy(k, carry):
        pltpu.sync_copy(rows_a, acc_sh.at[pl.ds(base + k * K * 128, K * 128)])
        return carry
    lax.fori_loop(0, RPS // (K * 128), _zcopy, 0)
    zt = (RPS // (K * 128)) * K * 128
    pltpu.sync_copy(rows_a.at[pl.ds(0, RPS - zt)],
                    acc_sh.at[pl.ds(base + zt, RPS - zt)])
    plsc.subcore_barrier()

    r0, cnt = _worker_rows(wid)

    def _idx(i, idx_s, idx_d):
        rb = r0 + i * K
        pltpu.async_copy(edge3.at[0].at[pl.ds(rb, K)], idx_s, si)
        pltpu.async_copy(edge3.at[1].at[pl.ds(rb, K)], idx_d, si)

    def _gath(idx_s, rows):
        for j in range(K):
            pltpu.async_copy(table.at[idx_s.at[j]],
                             rows.at[pl.ds(j * 128, 128)], sg)

    def _fire(i, idx_s, idx_d, rows):
        rb = r0 + i * K
        pltpu.sync_copy(edge3.at[0].at[pl.ds(rb, K)], idx_s)
        pltpu.sync_copy(edge3.at[1].at[pl.ds(rb, K)], idx_d)
        _gath(idx_s, rows)

    def _drain(sem, nstreams):
        if sem is si:
            for j in range(nstreams):
                pltpu.make_async_copy(edge3.at[0].at[pl.ds(0, K)],
                                      idx_sa, sem).wait()
        else:
            for j in range(nstreams):
                pltpu.make_async_copy(table.at[pl.ds(0, 128)],
                                      rows_a.at[pl.ds(0, 128)], sem).wait()

    def _scat(idx_d, rows):
        for j in range(K):
            pltpu.async_copy(rows.at[pl.ds(j * 128, 128)],
                             acc_sh.at[idx_d.at[j]], ss, add=True)

    _fire(0, idx_sa, idx_da, rows_a)
    def _pair(t, carry):
        @pl.when(t > 0)
        def _():
            _drain(ss, K)
        _idx(2 * t + 1, idx_sb, idx_db)
        _drain(sg, K)
        _scat(idx_da, rows_a)
        _drain(si, 2)
        _gath(idx_sb, rows_b)
        _drain(ss, K)
        @pl.when(t < F // 2 - 1)
        def _():
            _idx(2 * t + 2, idx_sa, idx_da)
        _drain(sg, K)
        _scat(idx_db, rows_b)
        @pl.when(t < F // 2 - 1)
        def _():
            _drain(si, 2)
            _gath(idx_sa, rows_a)
        return carry
    lax.fori_loop(0, F // 2, _pair, 0)
    _drain(ss, K)

    def _tail(t, carry):
        r = r0 + t
        pltpu.sync_copy(edge3.at[0].at[pl.ds(r, 1)], idx_sa.at[pl.ds(0, 1)])
        pltpu.sync_copy(edge3.at[1].at[pl.ds(r, 1)], idx_da.at[pl.ds(0, 1)])
        pltpu.async_copy(table.at[idx_sa.at[0]],
                         rows_a.at[pl.ds(0, 128)], sg).wait()
        pltpu.sync_copy(rows_a.at[pl.ds(0, 128)],
                        acc_sh.at[idx_da.at[0]], add=True)
        return carry
    lax.fori_loop(K * F, cnt, _tail, 0)
    plsc.subcore_barrier()

    def _wb(h, carry):
        pltpu.sync_copy(acc_sh.at[pl.ds(base + h * WCH, WCH)],
                        rows_a.at[pl.ds(0, WCH)])
        def _rep(w, carry2):
            for n in range(8):
                wbuf[w, pl.ds(16 * n, 16)] = rows_a[8 * w + n, :]
            return carry2
        lax.fori_loop(0, WCH // 8, _rep, 0)
        pltpu.sync_copy(wbuf,
                        out_acc.at[c].at[pl.ds(s * WPS + h * (WCH // 8),
                                               WCH // 8)])
        return carry
    lax.fori_loop(0, RPS // WCH, _wb, 0)


def _sc_deg_body(edge3, out_degw, deg_sh, idx_a, idx_b, ones_v, z1,
                 dtile, dbuf, ss):
    c = lax.axis_index("c")
    s = lax.axis_index("s")
    wid = s * NCORE + c

    def _zfill(i, carry):
        z1[pl.ds(i * 16, 16)] = jnp.zeros((16,), jnp.float32)
        return carry
    lax.fori_loop(0, ZR, _zfill, 0)
    base = s * RPS
    pltpu.sync_copy(z1, deg_sh.at[pl.ds(base, RPS)])
    for k in range(8):
        ones_v[pl.ds(k * 16, 16)] = jnp.ones((16,), jnp.float32)
    plsc.subcore_barrier()

    r0, cnt = _worker_rows(wid)

    def _load(i, idx):
        pltpu.sync_copy(edge3.at[1].at[pl.ds(r0 + i * K, K)], idx)

    def _scat(idx):
        for j in range(K):
            pltpu.async_copy(ones_v, deg_sh.at[idx.at[j]], ss, add=True)

    def _drain(n):
        for j in range(n):
            pltpu.make_async_copy(out_degw.at[c].at[0],
                                  ones_v, ss).wait()

    _load(0, idx_a)
    def _pair(t, carry):
        _scat(idx_a)
        @pl.when(t > 0)
        def _():
            _drain(K)
        _load(2 * t + 1, idx_b)
        _scat(idx_b)
        _drain(K)
        @pl.when(t < F // 2 - 1)
        def _():
            _load(2 * t + 2, idx_a)
        return carry
    lax.fori_loop(0, F // 2, _pair, 0)
    _drain(K)

    def _tail(t, carry):
        pltpu.sync_copy(edge3.at[1].at[pl.ds(r0 + t, 1)],
                        idx_a.at[pl.ds(0, 1)])
        pltpu.sync_copy(ones_v, deg_sh.at[idx_a.at[0]], add=True)
        return carry
    lax.fori_loop(K * F, cnt, _tail, 0)
    plsc.subcore_barrier()

    pltpu.sync_copy(deg_sh.at[pl.ds(base, RPS)], dtile)

    def _fill(roff, ng):
        def _g(g, carry2):
            v = dtile[pl.ds(roff + g * 16, 16)]
            for n in range(8):
                dbuf[2 * g, pl.ds(16 * n, 16)] = jnp.full((16,), v[n],
                                                          jnp.float32)
            for n in range(8):
                dbuf[2 * g + 1, pl.ds(16 * n, 16)] = jnp.full(
                    (16,), v[8 + n], jnp.float32)
            return carry2
        lax.fori_loop(0, ng, _g, 0)

    def _bq(q, carry):
        _fill(q * CB, CB // 16)
        pltpu.sync_copy(dbuf, out_degw.at[c].at[pl.ds(s * WPS + q * (CB // 8),
                                                      CB // 8)])
        return carry
    lax.fori_loop(0, 17, _bq, 0)
    tb = 17 * CB
    _fill(tb, (RPS - tb) // 16)
    pltpu.sync_copy(dbuf.at[pl.ds(0, (RPS - tb) // 8)],
                    out_degw.at[c].at[pl.ds(s * WPS + tb // 8,
                                            (RPS - tb) // 8)])


_MESH = plsc.VectorSubcoreMesh(core_axis_name="c", subcore_axis_name="s")
_SC_PARAMS = pltpu.CompilerParams(use_tc_tiling_on_sc=False)

_agg = pl.kernel(
    _sc_agg_body,
    compiler_params=_SC_PARAMS,
    out_type=jax.ShapeDtypeStruct((NCORE, WROWS, 128), jnp.float32),
    mesh=_MESH,
    scratch_types=[
        pltpu.VMEM_SHARED((NPAD, D), jnp.float32),
        pltpu.VMEM((K, 128), jnp.int32),
        pltpu.VMEM((K, 128), jnp.int32),
        pltpu.VMEM((K, 128), jnp.int32),
        pltpu.VMEM((K, 128), jnp.int32),
        pltpu.VMEM((K * 128, D), jnp.float32),
        pltpu.VMEM((K * 128, D), jnp.float32),
        pltpu.VMEM((WCH // 8, 128), jnp.float32),
        pltpu.SemaphoreType.DMA,
        pltpu.SemaphoreType.DMA,
        pltpu.SemaphoreType.DMA,
    ],
)

_deg_count = pl.kernel(
    _sc_deg_body,
    compiler_params=_SC_PARAMS,
    out_type=jax.ShapeDtypeStruct((NCORE, WROWS, 128), jnp.float32),
    mesh=_MESH,
    scratch_types=[
        pltpu.VMEM_SHARED((NPAD,), jnp.float32),
        pltpu.VMEM((K, 128), jnp.int32),
        pltpu.VMEM((K, 128), jnp.int32),
        pltpu.VMEM((128,), jnp.float32),
        pltpu.VMEM((RPS,), jnp.float32),
        pltpu.VMEM((RPS,), jnp.float32),
        pltpu.VMEM((CB // 8, 128), jnp.float32),
        pltpu.SemaphoreType.DMA,
    ],
)

RB = 3128
G = WROWS // RB


def _dense1_body(p_ref, degw_ref, x_ref, w1l_ref, b1_ref, w1r_ref,
                 w2l_ref, w2r_ref, b2_ref, g_ref, r_ref):
    p = p_ref[0] + p_ref[1]
    d = degw_ref[0] + degw_ref[1]
    mean = p / jnp.clip(d, 1.0)
    h = jnp.maximum(
        mean @ w1l_ref[...] + b1_ref[...] + x_ref[...] @ w1r_ref[...], 0.0)
    g_ref[...] = h @ w2l_ref[...]
    r_ref[...] = h @ w2r_ref[...] + b2_ref[...]


_dense1 = pl.pallas_call(
    _dense1_body,
    grid=(G,),
    in_specs=[
        pl.BlockSpec((NCORE, RB, 128), lambda i: (0, i, 0)),
        pl.BlockSpec((NCORE, RB, 128), lambda i: (0, i, 0)),
        pl.BlockSpec((RB, 128), lambda i: (i, 0)),
        pl.BlockSpec((128, 8 * H), lambda i: (0, 0)),
        pl.BlockSpec((1, 8 * H), lambda i: (0, 0)),
        pl.BlockSpec((128, 8 * H), lambda i: (0, 0)),
        pl.BlockSpec((8 * H, 128), lambda i: (0, 0)),
        pl.BlockSpec((8 * H, 128), lambda i: (0, 0)),
        pl.BlockSpec((1, 128), lambda i: (0, 0)),
    ],
    out_specs=[
        pl.BlockSpec((RB, 128), lambda i: (i, 0)),
        pl.BlockSpec((RB, 128), lambda i: (i, 0)),
    ],
    out_shape=[
        jax.ShapeDtypeStruct((WROWS, 128), jnp.float32),
        jax.ShapeDtypeStruct((WROWS, 128), jnp.float32),
    ],
)


def _dense2_body(p_ref, degw_ref, r_ref, o_ref):
    p = p_ref[0] + p_ref[1]
    d = degw_ref[0] + degw_ref[1]
    o_ref[...] = p / jnp.clip(d, 1.0) + r_ref[...]


_dense2 = pl.pallas_call(
    _dense2_body,
    grid=(G,),
    in_specs=[
        pl.BlockSpec((NCORE, RB, 128), lambda i: (0, i, 0)),
        pl.BlockSpec((NCORE, RB, 128), lambda i: (0, i, 0)),
        pl.BlockSpec((RB, 128), lambda i: (i, 0)),
    ],
    out_specs=pl.BlockSpec((RB, 128), lambda i: (i, 0)),
    out_shape=jax.ShapeDtypeStruct((WROWS, 128), jnp.float32),
)


def kernel(x, edge_index, W1_l, b1_l, W1_r, W2_l, b2_l, W2_r):
    f32 = jnp.float32
    edge3 = edge_index.reshape(2, EROWS, 128)
    xw = jnp.pad(x.reshape(E // 128, 128), ((0, WROWS - E // 128), (0, 0)))

    eye8 = jnp.eye(8, dtype=f32)
    w1l_w = jnp.kron(eye8, W1_l)
    w1r_w = jnp.kron(eye8, W1_r)
    w2l_w = jnp.kron(eye8, W2_l)
    w2r_w = jnp.kron(eye8, W2_r)
    b1_w = jnp.tile(b1_l, 8).reshape(1, 8 * H)
    b2_w = jnp.tile(b2_l, 8).reshape(1, 128)

    p1 = _agg(x, edge3)
    degw = _deg_count(edge3)
    gw, rw = _dense1(p1, degw, xw, w1l_w, b1_w, w1r_w, w2l_w, w2r_w, b2_w)
    p2 = _agg(gw.reshape(NPAD, D), edge3)
    outw = _dense2(p2, degw, rw)
    return outw[:E // 128].reshape(N, D)


# speedup vs baseline: 34.7329x; 1.0018x over previous
import jax
import jax.numpy as jnp
from jax import lax
from jax.experimental import pallas as pl
from jax.experimental.pallas import tpu as pltpu
from jax.experimental.pallas import tpu_sc as plsc

N = 100000
E = 1600000
D = 16
H = 32

NSUB = 16
NCORE = 2
NW = NSUB * NCORE
NPAD = 100096
RPS = NPAD // NSUB
WROWS = NPAD // 8
WPS = RPS // 8
EROWS = E // 128
K = 5
F = 78
ZR = RPS // NSUB
CB = 352
WCH = 368


def _worker_rows(wid):
    r0 = 390 * wid + jnp.minimum(wid, 20)
    cnt = jnp.where(wid < 20, 391, 390)
    return r0, cnt


def _sc_agg_body(table, edge3, out_acc,
                 acc_sh, idx_sa, idx_da, idx_sb, idx_db,
                 rows_a, rows_b, wbuf, sg, ss, si):
    c = lax.axis_index("c")
    s = lax.axis_index("s")
    wid = s * NCORE + c

    def _zfill(i, carry):
        rows_a[i, :] = jnp.zeros((16,), jnp.float32)
        return carry
    lax.fori_loop(0, K * 128, _zfill, 0)
    base = s * RPS
    def _zcopy(k, carry):
        pltpu.sync_copy(rows_a, acc_sh.at[pl.ds(base + k * K * 128, K * 128)])
        return carry
    lax.fori_loop(0, RPS // (K * 128), _zcopy, 0)
    zt = (RPS // (K * 128)) * K * 128
    pltpu.sync_copy(rows_a.at[pl.ds(0, RPS - zt)],
                    acc_sh.at[pl.ds(base + zt, RPS - zt)])
    plsc.subcore_barrier()

    r0, cnt = _worker_rows(wid)

    def _idx(i, idx_s, idx_d):
        rb = r0 + i * K
        pltpu.async_copy(edge3.at[0].at[pl.ds(rb, K)], idx_s, si)
        pltpu.async_copy(edge3.at[1].at[pl.ds(rb, K)], idx_d, si)

    def _gath(idx_s, rows):
        for j in range(K):
            pltpu.async_copy(table.at[idx_s.at[j]],
                             rows.at[pl.ds(j * 128, 128)], sg)

    def _fire(i, idx_s, idx_d, rows):
        rb = r0 + i * K
        pltpu.sync_copy(edge3.at[0].at[pl.ds(rb, K)], idx_s)
        pltpu.sync_copy(edge3.at[1].at[pl.ds(rb, K)], idx_d)
        _gath(idx_s, rows)

    def _drain(sem, nstreams):
        if sem is si:
            for j in range(nstreams):
                pltpu.make_async_copy(edge3.at[0].at[pl.ds(0, K)],
                                      idx_sa, sem).wait()
        else:
            for j in range(nstreams):
                pltpu.make_async_copy(table.at[pl.ds(0, 128)],
                                      rows_a.at[pl.ds(0, 128)], sem).wait()

    def _scat(idx_d, rows):
        for j in range(K):
            pltpu.async_copy(rows.at[pl.ds(j * 128, 128)],
                             acc_sh.at[idx_d.at[j]], ss, add=True)

    _fire(0, idx_sa, idx_da, rows_a)
    def _pair(t, carry):
        @pl.when(t > 0)
        def _():
            _drain(ss, K)
        _idx(2 * t + 1, idx_sb, idx_db)
        _drain(sg, K)
        _scat(idx_da, rows_a)
        _drain(si, 2)
        _gath(idx_sb, rows_b)
        _drain(ss, K)
        @pl.when(t < F // 2 - 1)
        def _():
            _idx(2 * t + 2, idx_sa, idx_da)
        _drain(sg, K)
        _scat(idx_db, rows_b)
        @pl.when(t < F // 2 - 1)
        def _():
            _drain(si, 2)
            _gath(idx_sa, rows_a)
        return carry
    lax.fori_loop(0, F // 2, _pair, 0)
    _drain(ss, K)

    def _tail(t, carry):
        r = r0 + t
        pltpu.sync_copy(edge3.at[0].at[pl.ds(r, 1)], idx_sa.at[pl.ds(0, 1)])
        pltpu.sync_copy(edge3.at[1].at[pl.ds(r, 1)], idx_da.at[pl.ds(0, 1)])
        pltpu.async_copy(table.at[idx_sa.at[0]],
                         rows_a.at[pl.ds(0, 128)], sg).wait()
        pltpu.sync_copy(rows_a.at[pl.ds(0, 128)],
                        acc_sh.at[idx_da.at[0]], add=True)
        return carry
    lax.fori_loop(K * F, cnt, _tail, 0)
    plsc.subcore_barrier()

    def _wb(h, carry):
        pltpu.sync_copy(acc_sh.at[pl.ds(base + h * WCH, WCH)],
                        rows_a.at[pl.ds(0, WCH)])
        def _rep(w, carry2):
            for n in range(8):
                wbuf[w, pl.ds(16 * n, 16)] = rows_a[8 * w + n, :]
            return carry2
        lax.fori_loop(0, WCH // 8, _rep, 0)
        pltpu.sync_copy(wbuf,
                        out_acc.at[c].at[pl.ds(s * WPS + h * (WCH // 8),
                                               WCH // 8)])
        return carry
    lax.fori_loop(0, RPS // WCH, _wb, 0)


def _sc_deg_body(edge3, out_degw, deg_sh, idx_a, idx_b, ones_v, z1,
                 dtile, dbuf, ss, si):
    c = lax.axis_index("c")
    s = lax.axis_index("s")
    wid = s * NCORE + c

    def _zfill(i, carry):
        z1[pl.ds(i * 16, 16)] = jnp.zeros((16,), jnp.float32)
        return carry
    lax.fori_loop(0, ZR, _zfill, 0)
    base = s * RPS
    pltpu.sync_copy(z1, deg_sh.at[pl.ds(base, RPS)])
    for k in range(8):
        ones_v[pl.ds(k * 16, 16)] = jnp.ones((16,), jnp.float32)
    plsc.subcore_barrier()

    r0, cnt = _worker_rows(wid)

    def _load(i, idx):
        pltpu.sync_copy(edge3.at[1].at[pl.ds(r0 + i * K, K)], idx)

    def _loada(i, idx):
        pltpu.async_copy(edge3.at[1].at[pl.ds(r0 + i * K, K)], idx, si)

    def _scat(idx):
        for j in range(K):
            pltpu.async_copy(ones_v, deg_sh.at[idx.at[j]], ss, add=True)

    def _drain(n):
        for j in range(n):
            pltpu.make_async_copy(out_degw.at[c].at[0],
                                  ones_v, ss).wait()

    def _drain_si(n):
        for j in range(n):
            pltpu.make_async_copy(edge3.at[1].at[pl.ds(0, K)],
                                  idx_a, si).wait()

    _load(0, idx_a)
    def _pair(t, carry):
        @pl.when(t > 0)
        def _():
            _drain_si(1)
        _scat(idx_a)
        @pl.when(t > 0)
        def _():
            _drain(K)
        _loada(2 * t + 1, idx_b)
        _drain(K)
        _drain_si(1)
        _scat(idx_b)
        @pl.when(t < F // 2 - 1)
        def _():
            _loada(2 * t + 2, idx_a)
        return carry
    lax.fori_loop(0, F // 2, _pair, 0)
    _drain(K)

    def _tail(t, carry):
        pltpu.sync_copy(edge3.at[1].at[pl.ds(r0 + t, 1)],
                        idx_a.at[pl.ds(0, 1)])
        pltpu.sync_copy(ones_v, deg_sh.at[idx_a.at[0]], add=True)
        return carry
    lax.fori_loop(K * F, cnt, _tail, 0)
    plsc.subcore_barrier()

    pltpu.sync_copy(deg_sh.at[pl.ds(base, RPS)], dtile)

    def _fill(roff, ng):
        def _g(g, carry2):
            v = dtile[pl.ds(roff + g * 16, 16)]
            for n in range(8):
                dbuf[2 * g, pl.ds(16 * n, 16)] = jnp.full((16,), v[n],
                                                          jnp.float32)
            for n in range(8):
                dbuf[2 * g + 1, pl.ds(16 * n, 16)] = jnp.full(
                    (16,), v[8 + n], jnp.float32)
            return carry2
        lax.fori_loop(0, ng, _g, 0)

    def _bq(q, carry):
        _fill(q * CB, CB // 16)
        pltpu.sync_copy(dbuf, out_degw.at[c].at[pl.ds(s * WPS + q * (CB // 8),
                                                      CB // 8)])
        return carry
    lax.fori_loop(0, 17, _bq, 0)
    tb = 17 * CB
    _fill(tb, (RPS - tb) // 16)
    pltpu.sync_copy(dbuf.at[pl.ds(0, (RPS - tb) // 8)],
                    out_degw.at[c].at[pl.ds(s * WPS + tb // 8,
                                            (RPS - tb) // 8)])


_MESH = plsc.VectorSubcoreMesh(core_axis_name="c", subcore_axis_name="s")
_SC_PARAMS = pltpu.CompilerParams(use_tc_tiling_on_sc=False)

_agg = pl.kernel(
    _sc_agg_body,
    compiler_params=_SC_PARAMS,
    out_type=jax.ShapeDtypeStruct((NCORE, WROWS, 128), jnp.float32),
    mesh=_MESH,
    scratch_types=[
        pltpu.VMEM_SHARED((NPAD, D), jnp.float32),
        pltpu.VMEM((K, 128), jnp.int32),
        pltpu.VMEM((K, 128), jnp.int32),
        pltpu.VMEM((K, 128), jnp.int32),
        pltpu.VMEM((K, 128), jnp.int32),
        pltpu.VMEM((K * 128, D), jnp.float32),
        pltpu.VMEM((K * 128, D), jnp.float32),
        pltpu.VMEM((WCH // 8, 128), jnp.float32),
        pltpu.SemaphoreType.DMA,
        pltpu.SemaphoreType.DMA,
        pltpu.SemaphoreType.DMA,
    ],
)

_deg_count = pl.kernel(
    _sc_deg_body,
    compiler_params=_SC_PARAMS,
    out_type=jax.ShapeDtypeStruct((NCORE, WROWS, 128), jnp.float32),
    mesh=_MESH,
    scratch_types=[
        pltpu.VMEM_SHARED((NPAD,), jnp.float32),
        pltpu.VMEM((K, 128), jnp.int32),
        pltpu.VMEM((K, 128), jnp.int32),
        pltpu.VMEM((128,), jnp.float32),
        pltpu.VMEM((RPS,), jnp.float32),
        pltpu.VMEM((RPS,), jnp.float32),
        pltpu.VMEM((CB // 8, 128), jnp.float32),
        pltpu.SemaphoreType.DMA,
        pltpu.SemaphoreType.DMA,
    ],
)

RB = 3128
G = WROWS // RB


def _dense1_body(p_ref, degw_ref, x_ref, w1l_ref, b1_ref, w1r_ref,
                 w2l_ref, w2r_ref, b2_ref, g_ref, r_ref):
    p = p_ref[0] + p_ref[1]
    d = degw_ref[0] + degw_ref[1]
    mean = p / jnp.clip(d, 1.0)
    h = jnp.maximum(
        mean @ w1l_ref[...] + b1_ref[...] + x_ref[...] @ w1r_ref[...], 0.0)
    g_ref[...] = h @ w2l_ref[...]
    r_ref[...] = h @ w2r_ref[...] + b2_ref[...]


_dense1 = pl.pallas_call(
    _dense1_body,
    grid=(G,),
    in_specs=[
        pl.BlockSpec((NCORE, RB, 128), lambda i: (0, i, 0)),
        pl.BlockSpec((NCORE, RB, 128), lambda i: (0, i, 0)),
        pl.BlockSpec((RB, 128), lambda i: (i, 0)),
        pl.BlockSpec((128, 8 * H), lambda i: (0, 0)),
        pl.BlockSpec((1, 8 * H), lambda i: (0, 0)),
        pl.BlockSpec((128, 8 * H), lambda i: (0, 0)),
        pl.BlockSpec((8 * H, 128), lambda i: (0, 0)),
        pl.BlockSpec((8 * H, 128), lambda i: (0, 0)),
        pl.BlockSpec((1, 128), lambda i: (0, 0)),
    ],
    out_specs=[
        pl.BlockSpec((RB, 128), lambda i: (i, 0)),
        pl.BlockSpec((RB, 128), lambda i: (i, 0)),
    ],
    out_shape=[
        jax.ShapeDtypeStruct((WROWS, 128), jnp.float32),
        jax.ShapeDtypeStruct((WROWS, 128), jnp.float32),
    ],
)


def _dense2_body(p_ref, degw_ref, r_ref, o_ref):
    p = p_ref[0] + p_ref[1]
    d = degw_ref[0] + degw_ref[1]
    o_ref[...] = p / jnp.clip(d, 1.0) + r_ref[...]


_dense2 = pl.pallas_call(
    _dense2_body,
    grid=(G,),
    in_specs=[
        pl.BlockSpec((NCORE, RB, 128), lambda i: (0, i, 0)),
        pl.BlockSpec((NCORE, RB, 128), lambda i: (0, i, 0)),
        pl.BlockSpec((RB, 128), lambda i: (i, 0)),
    ],
    out_specs=pl.BlockSpec((RB, 128), lambda i: (i, 0)),
    out_shape=jax.ShapeDtypeStruct((WROWS, 128), jnp.float32),
)


def kernel(x, edge_index, W1_l, b1_l, W1_r, W2_l, b2_l, W2_r):
    f32 = jnp.float32
    edge3 = edge_index.reshape(2, EROWS, 128)
    xw = jnp.pad(x.reshape(E // 128, 128), ((0, WROWS - E // 128), (0, 0)))

    eye8 = jnp.eye(8, dtype=f32)
    w1l_w = jnp.kron(eye8, W1_l)
    w1r_w = jnp.kron(eye8, W1_r)
    w2l_w = jnp.kron(eye8, W2_l)
    w2r_w = jnp.kron(eye8, W2_r)
    b1_w = jnp.tile(b1_l, 8).reshape(1, 8 * H)
    b2_w = jnp.tile(b2_l, 8).reshape(1, 128)

    p1 = _agg(x, edge3)
    degw = _deg_count(edge3)
    gw, rw = _dense1(p1, degw, xw, w1l_w, b1_w, w1r_w, w2l_w, w2r_w, b2_w)
    p2 = _agg(gw.reshape(NPAD, D), edge3)
    outw = _dense2(p2, degw, rw)
    return outw[:E // 128].reshape(N, D)
